# baseline (speedup 1.0000x reference)
import jax
import jax.numpy as jnp
from jax import lax
from jax.experimental import pallas as pl
from jax.experimental.pallas import tpu as pltpu
from jax.experimental.pallas import tpu_sc as plsc

_N = 4096
_D = 64
_E = 262144
_NC = 2
_NS = 16
_NW = _NC * _NS
_EPW = _E // _NW
_S1 = 256
_S2 = 512
_BM = 256


def _mm_body(w_ref, x_ref, o_ref):
    o_ref[...] = jnp.dot(w_ref[...], x_ref[...],
                         preferred_element_type=jnp.float32)


def _matmul_af(W, act):
    return pl.pallas_call(
        _mm_body,
        grid=(_N // _BM,),
        in_specs=[
            pl.BlockSpec((_BM, _N), lambda i: (i, 0)),
            pl.BlockSpec((_N, _D), lambda i: (0, 0)),
        ],
        out_specs=pl.BlockSpec((_BM, _D), lambda i: (i, 0)),
        out_shape=jax.ShapeDtypeStruct((_N, _D), jnp.float32),
    )(W, act)


def _combine_body(num_ref, den_ref, af_ref, o_ref):
    den = den_ref[0, :] + den_ref[1, :]
    num = num_ref[0] + num_ref[1]
    o_ref[...] = num / (den[:, None] + 1e-12) + af_ref[...]


def _combine(num2, den2, af):
    blk = 512
    return pl.pallas_call(
        _combine_body,
        grid=(_N // blk,),
        in_specs=[
            pl.BlockSpec((2, blk, _D), lambda i: (0, i, 0)),
            pl.BlockSpec((2, blk), lambda i: (0, i)),
            pl.BlockSpec((blk, _D), lambda i: (i, 0)),
        ],
        out_specs=pl.BlockSpec((blk, _D), lambda i: (i, 0)),
        out_shape=jax.ShapeDtypeStruct((_N, _D), jnp.float32),
    )(num2, den2, af)


def _rsqrt_newton(ss):
    i = plsc.bitcast(ss, jnp.int32)
    i = jnp.int32(0x5F3759DF) - (i >> 1)
    y = plsc.bitcast(i, jnp.float32)
    for _ in range(3):
        y = y * (1.5 - 0.5 * ss * y * y)
    return y


def _stage1_body(c2, t2, act, cases, v_out,
                 cv, tv, vv, hc_a, ht_a, cs_a, hc_b, ht_b, cs_b, tr,
                 sem_a, sem_b):
    cid = lax.axis_index("c")
    sid = lax.axis_index("s")
    wid = sid * _NC + cid
    ebase = pl.multiple_of(wid * _EPW, _EPW)
    rbase0 = pl.multiple_of(wid * (_EPW // 128), 8)
    col16 = lax.iota(jnp.int32, 16)
    nsub = _EPW // _S1
    rps = _S1 // 128
    pltpu.sync_copy(c2.at[pl.ds(rbase0, _EPW // 128)], cv)
    pltpu.sync_copy(t2.at[pl.ds(rbase0, _EPW // 128)], tv)

    def copies(s, hc, ht, cs, sem, fire):
        trips = []
        for j in range(rps):
            r = s * rps + j
            trips.append((act.at[cv.at[r]], hc.at[pl.ds(j * 128, 128)], sem))
            trips.append((act.at[tv.at[r]], ht.at[pl.ds(j * 128, 128)], sem))
        off = pl.multiple_of(ebase + s * _S1, _S1)
        trips.append((cases.at[pl.ds(off, _S1)], cs, sem))
        for src, dst, sm in trips:
            if fire:
                pltpu.async_copy(src, dst, sm)
            else:
                pltpu.make_async_copy(src, dst, sm).wait()

    col17 = col16 * 17

    def compute(s, hc, ht, cs):
        def group_body(g, _):
            for j in range(16):
                e = g * 16 + j
                sq = None
                for k in range(4):
                    sl = pl.ds(k * 16, 16)
                    df = hc[e, sl] + cs[e, sl] - ht[e, sl]
                    p = df * df
                    sq = p if sq is None else sq + p
                tr[pl.ds(j * 17, 16)] = sq
            tots = [jnp.zeros((16,), jnp.float32) for _ in range(4)]
            for i in range(16):
                tots[i % 4] = tots[i % 4] + plsc.load_gather(tr, [col17 + i])
            acc = (tots[0] + tots[1]) + (tots[2] + tots[3])
            ss = jnp.maximum(acc, 1e-30)
            dist = acc * _rsqrt_newton(ss)
            vv[pl.ds(s * _S1 + g * 16, 16)] = jnp.exp(-dist)
            return 0

        lax.fori_loop(0, _S1 // 16, group_body, 0)

    copies(0, hc_a, ht_a, cs_a, sem_a, True)

    def pipe_body(i, _):
        copies(2 * i + 1, hc_b, ht_b, cs_b, sem_b, True)
        copies(2 * i, hc_a, ht_a, cs_a, sem_a, False)
        compute(2 * i, hc_a, ht_a, cs_a)

        @pl.when(i < nsub // 2 - 1)
        def _():
            copies(2 * i + 2, hc_a, ht_a, cs_a, sem_a, True)

        copies(2 * i + 1, hc_b, ht_b, cs_b, sem_b, False)
        compute(2 * i + 1, hc_b, ht_b, cs_b)
        return 0

    lax.fori_loop(0, nsub // 2, pipe_body, 0)
    pltpu.sync_copy(vv, v_out.at[pl.ds(ebase, _EPW)])


def _stage2_body(c2, t2, v2, af, z2d, z1d, num_out, den_out,
                 cv, tv, vvb, rows_a, rows_b, num_sh, den_sh, af_sh,
                 sem_a, sem_b, sem_sa, sem_sb):
    cid = lax.axis_index("c")
    sid = lax.axis_index("s")
    wid = sid * _NC + cid
    ebase = pl.multiple_of(wid * _EPW, _EPW)
    rbase0 = pl.multiple_of(wid * (_EPW // 128), 8)
    npc = _N // _NS
    srow = pl.multiple_of(sid * npc, 8)
    nsub = _EPW // _S2
    rps = _S2 // 128

    pltpu.sync_copy(z2d.at[pl.ds(srow, npc)], num_sh.at[pl.ds(srow, npc)])
    pltpu.sync_copy(z1d.at[pl.ds(srow, npc)], den_sh.at[pl.ds(srow, npc)])
    pltpu.sync_copy(af.at[pl.ds(srow, npc)], af_sh.at[pl.ds(srow, npc)])

    pltpu.sync_copy(c2.at[pl.ds(rbase0, _EPW // 128)], cv)
    pltpu.sync_copy(t2.at[pl.ds(rbase0, _EPW // 128)], tv)
    pltpu.sync_copy(v2.at[pl.ds(rbase0, _EPW // 128)], vvb)
    plsc.subcore_barrier()

    def gathers(s, rows, sem, fire):
        for j in range(rps):
            src = af_sh.at[cv.at[s * rps + j]]
            dst = rows.at[pl.ds(j * 128, 128)]
            if fire:
                pltpu.async_copy(src, dst, sem)
            else:
                pltpu.make_async_copy(src, dst, sem).wait()

    def scatters(s, rows, sem, fire):
        for j in range(rps):
            r = s * rps + j
            pairs = [(rows.at[pl.ds(j * 128, 128)], num_sh.at[tv.at[r]]),
                     (vvb.at[r], den_sh.at[tv.at[r]])]
            for src, dst in pairs:
                if fire:
                    pltpu.async_copy(src, dst, sem, add=True)
                else:
                    pltpu.make_async_copy(src, dst, sem).wait()

    def scale(s, rows):
        def scale_body(g, _):
            v16 = vvb[s * rps + g // 8, pl.ds((g % 8) * 16, 16)]
            for j in range(16):
                e = g * 16 + j
                vb = jnp.broadcast_to(v16[j], (16,))
                for k in range(4):
                    sl = pl.ds(k * 16, 16)
                    rows[e, sl] = rows[e, sl] * vb
            return 0

        lax.fori_loop(0, _S2 // 16, scale_body, 0)

    gathers(0, rows_a, sem_a, True)

    def pipe_body(i, _):
        @pl.when(i > 0)
        def _():
            scatters(2 * i - 1, rows_b, sem_sb, False)

        gathers(2 * i + 1, rows_b, sem_b, True)
        gathers(2 * i, rows_a, sem_a, False)
        scale(2 * i, rows_a)
        scatters(2 * i, rows_a, sem_sa, True)
        gathers(2 * i + 1, rows_b, sem_b, False)
        scale(2 * i + 1, rows_b)

        @pl.when(i < nsub // 2 - 1)
        def _():
            scatters(2 * i, rows_a, sem_sa, False)
            gathers(2 * i + 2, rows_a, sem_a, True)

        scatters(2 * i + 1, rows_b, sem_sb, True)
        return 0

    lax.fori_loop(0, nsub // 2, pipe_body, 0)
    scatters(nsub - 2, rows_a, sem_sa, False)
    scatters(nsub - 1, rows_b, sem_sb, False)
    plsc.subcore_barrier()
    pltpu.sync_copy(num_sh.at[pl.ds(srow, npc)],
                    num_out.at[cid, pl.ds(srow, npc)])
    pltpu.sync_copy(den_sh.at[pl.ds(srow, npc)],
                    den_out.at[cid, pl.ds(srow, npc)])


def _edge_vals(currents2, targets2, act, cases_flat):
    mesh = plsc.VectorSubcoreMesh(core_axis_name="c", subcore_axis_name="s")
    f = pl.kernel(
        _stage1_body,
        out_type=jax.ShapeDtypeStruct((_E,), jnp.float32),
        mesh=mesh,
        compiler_params=pltpu.CompilerParams(needs_layout_passes=False,
                                             use_tc_tiling_on_sc=False),
        scratch_types=[
            pltpu.VMEM((_EPW // 128, 128), jnp.int32),
            pltpu.VMEM((_EPW // 128, 128), jnp.int32),
            pltpu.VMEM((_EPW,), jnp.float32),
            pltpu.VMEM((_S1, _D), jnp.float32),
            pltpu.VMEM((_S1, _D), jnp.float32),
            pltpu.VMEM((_S1, _D), jnp.float32),
            pltpu.VMEM((_S1, _D), jnp.float32),
            pltpu.VMEM((_S1, _D), jnp.float32),
            pltpu.VMEM((_S1, _D), jnp.float32),
            pltpu.VMEM((16 * 17,), jnp.float32),
            pltpu.SemaphoreType.DMA,
            pltpu.SemaphoreType.DMA,
        ],
    )
    return f(currents2, targets2, act, cases_flat)


def _segment_sums(currents2, targets2, v2, af):
    mesh = plsc.VectorSubcoreMesh(core_axis_name="c", subcore_axis_name="s")
    z2d = jnp.zeros((_N, _D), jnp.float32)
    z1d = jnp.zeros((_N,), jnp.float32)
    f = pl.kernel(
        _stage2_body,
        out_type=(jax.ShapeDtypeStruct((_NC, _N, _D), jnp.float32),
                  jax.ShapeDtypeStruct((_NC, _N), jnp.float32)),
        mesh=mesh,
        compiler_params=pltpu.CompilerParams(needs_layout_passes=False,
                                             use_tc_tiling_on_sc=False),
        scratch_types=[
            pltpu.VMEM((_EPW // 128, 128), jnp.int32),
            pltpu.VMEM((_EPW // 128, 128), jnp.int32),
            pltpu.VMEM((_EPW // 128, 128), jnp.float32),
            pltpu.VMEM((_S2, _D), jnp.float32),
            pltpu.VMEM((_S2, _D), jnp.float32),
            pltpu.VMEM_SHARED((_N, _D), jnp.float32),
            pltpu.VMEM_SHARED((_N,), jnp.float32),
            pltpu.VMEM_SHARED((_N, _D), jnp.float32),
            pltpu.SemaphoreType.DMA,
            pltpu.SemaphoreType.DMA,
            pltpu.SemaphoreType.DMA,
            pltpu.SemaphoreType.DMA,
        ],
    )
    return f(currents2, targets2, v2, af, z2d, z1d)


def kernel(currents, targets, activities_features, cases_features, W):
    c2 = currents.reshape(_E // 128, 128)
    t2 = targets.reshape(_E // 128, 128)
    af = _matmul_af(W, activities_features)
    v = _edge_vals(c2, t2, activities_features, cases_features)
    num2, den2 = _segment_sums(c2, t2, v.reshape(_E // 128, 128), af)
    return _combine(num2, den2, af)


# speedup vs baseline: 1.0719x; 1.0719x over previous
import jax
import jax.numpy as jnp
from jax import lax
from jax.experimental import pallas as pl
from jax.experimental.pallas import tpu as pltpu
from jax.experimental.pallas import tpu_sc as plsc

_N = 4096
_D = 64
_E = 262144
_NC = 2
_NS = 16
_NW = _NC * _NS
_EPW = _E // _NW
_S1 = 128
_S2 = 512
_BM = 256


def _mm_body(w_ref, x_ref, o_ref):
    o_ref[...] = jnp.dot(w_ref[...], x_ref[...],
                         preferred_element_type=jnp.float32)


def _matmul_af(W, act):
    return pl.pallas_call(
        _mm_body,
        grid=(_N // _BM,),
        in_specs=[
            pl.BlockSpec((_BM, _N), lambda i: (i, 0)),
            pl.BlockSpec((_N, _D), lambda i: (0, 0)),
        ],
        out_specs=pl.BlockSpec((_BM, _D), lambda i: (i, 0)),
        out_shape=jax.ShapeDtypeStruct((_N, _D), jnp.float32),
    )(W, act)


def _combine_body(num_ref, den_ref, af_ref, o_ref):
    den = den_ref[0, :] + den_ref[1, :]
    num = num_ref[0] + num_ref[1]
    o_ref[...] = num / (den[:, None] + 1e-12) + af_ref[...]


def _combine(num2, den2, af):
    blk = 512
    return pl.pallas_call(
        _combine_body,
        grid=(_N // blk,),
        in_specs=[
            pl.BlockSpec((2, blk, _D), lambda i: (0, i, 0)),
            pl.BlockSpec((2, blk), lambda i: (0, i)),
            pl.BlockSpec((blk, _D), lambda i: (i, 0)),
        ],
        out_specs=pl.BlockSpec((blk, _D), lambda i: (i, 0)),
        out_shape=jax.ShapeDtypeStruct((_N, _D), jnp.float32),
    )(num2, den2, af)


def _rsqrt_newton(ss):
    i = plsc.bitcast(ss, jnp.int32)
    i = jnp.int32(0x5F3759DF) - (i >> 1)
    y = plsc.bitcast(i, jnp.float32)
    for _ in range(3):
        y = y * (1.5 - 0.5 * ss * y * y)
    return y


def _stage1_body(c2, t2, act, cases, v_out,
                 cv, tv, vv, hc_a, ht_a, cs_a, hc_b, ht_b, cs_b, tr,
                 sem_a, sem_b):
    cid = lax.axis_index("c")
    sid = lax.axis_index("s")
    wid = sid * _NC + cid
    ebase = pl.multiple_of(wid * _EPW, _EPW)
    rbase0 = pl.multiple_of(wid * (_EPW // 128), 8)
    col16 = lax.iota(jnp.int32, 16)
    nsub = _EPW // _S1
    rps = _S1 // 128
    pltpu.sync_copy(c2.at[pl.ds(rbase0, _EPW // 128)], cv)
    pltpu.sync_copy(t2.at[pl.ds(rbase0, _EPW // 128)], tv)

    def copies(s, hc, ht, cs, sem, fire):
        trips = []
        for j in range(rps):
            r = s * rps + j
            trips.append((act.at[cv.at[r]], hc.at[pl.ds(j * 128, 128)], sem))
            trips.append((act.at[tv.at[r]], ht.at[pl.ds(j * 128, 128)], sem))
        off = pl.multiple_of(ebase + s * _S1, _S1)
        trips.append((cases.at[pl.ds(off, _S1)], cs, sem))
        for src, dst, sm in trips:
            if fire:
                pltpu.async_copy(src, dst, sm)
            else:
                pltpu.make_async_copy(src, dst, sm).wait()

    col17 = col16 * 17

    def compute(s, hc, ht, cs):
        def group_body(g, _):
            for j in range(16):
                e = g * 16 + j
                sq = None
                for k in range(4):
                    sl = pl.ds(k * 16, 16)
                    df = hc[e, sl] + cs[e, sl] - ht[e, sl]
                    p = df * df
                    sq = p if sq is None else sq + p
                tr[pl.ds(j * 17, 16)] = sq
            tots = [jnp.zeros((16,), jnp.float32) for _ in range(4)]
            for i in range(16):
                tots[i % 4] = tots[i % 4] + plsc.load_gather(tr, [col17 + i])
            acc = (tots[0] + tots[1]) + (tots[2] + tots[3])
            ss = jnp.maximum(acc, 1e-30)
            dist = acc * _rsqrt_newton(ss)
            vv[pl.ds(s * _S1 + g * 16, 16)] = jnp.exp(-dist)
            return 0

        lax.fori_loop(0, _S1 // 16, group_body, 0)

    copies(0, hc_a, ht_a, cs_a, sem_a, True)

    def pipe_body(i, _):
        copies(2 * i + 1, hc_b, ht_b, cs_b, sem_b, True)
        copies(2 * i, hc_a, ht_a, cs_a, sem_a, False)
        compute(2 * i, hc_a, ht_a, cs_a)

        @pl.when(i < nsub // 2 - 1)
        def _():
            copies(2 * i + 2, hc_a, ht_a, cs_a, sem_a, True)

        copies(2 * i + 1, hc_b, ht_b, cs_b, sem_b, False)
        compute(2 * i + 1, hc_b, ht_b, cs_b)
        return 0

    lax.fori_loop(0, nsub // 2, pipe_body, 0)
    pltpu.sync_copy(vv, v_out.at[pl.ds(ebase, _EPW)])


def _stage2_body(c2, t2, v2, af, z2d, z1d, num_out, den_out,
                 cv, tv, vvb, rows_a, rows_b, num_sh, den_sh, af_sh,
                 sem_a, sem_b, sem_sa, sem_sb):
    cid = lax.axis_index("c")
    sid = lax.axis_index("s")
    wid = sid * _NC + cid
    ebase = pl.multiple_of(wid * _EPW, _EPW)
    rbase0 = pl.multiple_of(wid * (_EPW // 128), 8)
    npc = _N // _NS
    srow = pl.multiple_of(sid * npc, 8)
    nsub = _EPW // _S2
    rps = _S2 // 128

    pltpu.sync_copy(z2d.at[pl.ds(srow, npc)], num_sh.at[pl.ds(srow, npc)])
    pltpu.sync_copy(z1d.at[pl.ds(srow, npc)], den_sh.at[pl.ds(srow, npc)])
    pltpu.sync_copy(af.at[pl.ds(srow, npc)], af_sh.at[pl.ds(srow, npc)])

    pltpu.sync_copy(c2.at[pl.ds(rbase0, _EPW // 128)], cv)
    pltpu.sync_copy(t2.at[pl.ds(rbase0, _EPW // 128)], tv)
    pltpu.sync_copy(v2.at[pl.ds(rbase0, _EPW // 128)], vvb)
    plsc.subcore_barrier()

    def gathers(s, rows, sem, fire):
        for j in range(rps):
            src = af_sh.at[cv.at[s * rps + j]]
            dst = rows.at[pl.ds(j * 128, 128)]
            if fire:
                pltpu.async_copy(src, dst, sem)
            else:
                pltpu.make_async_copy(src, dst, sem).wait()

    def scatters(s, rows, sem, fire):
        for j in range(rps):
            r = s * rps + j
            pairs = [(rows.at[pl.ds(j * 128, 128)], num_sh.at[tv.at[r]]),
                     (vvb.at[r], den_sh.at[tv.at[r]])]
            for src, dst in pairs:
                if fire:
                    pltpu.async_copy(src, dst, sem, add=True)
                else:
                    pltpu.make_async_copy(src, dst, sem).wait()

    def scale(s, rows):
        def scale_body(g, _):
            v16 = vvb[s * rps + g // 8, pl.ds((g % 8) * 16, 16)]
            for j in range(16):
                e = g * 16 + j
                vb = jnp.broadcast_to(v16[j], (16,))
                for k in range(4):
                    sl = pl.ds(k * 16, 16)
                    rows[e, sl] = rows[e, sl] * vb
            return 0

        lax.fori_loop(0, _S2 // 16, scale_body, 0)

    gathers(0, rows_a, sem_a, True)

    def pipe_body(i, _):
        @pl.when(i > 0)
        def _():
            scatters(2 * i - 1, rows_b, sem_sb, False)

        gathers(2 * i + 1, rows_b, sem_b, True)
        gathers(2 * i, rows_a, sem_a, False)
        scale(2 * i, rows_a)
        scatters(2 * i, rows_a, sem_sa, True)
        gathers(2 * i + 1, rows_b, sem_b, False)
        scale(2 * i + 1, rows_b)

        @pl.when(i < nsub // 2 - 1)
        def _():
            scatters(2 * i, rows_a, sem_sa, False)
            gathers(2 * i + 2, rows_a, sem_a, True)

        scatters(2 * i + 1, rows_b, sem_sb, True)
        return 0

    lax.fori_loop(0, nsub // 2, pipe_body, 0)
    scatters(nsub - 2, rows_a, sem_sa, False)
    scatters(nsub - 1, rows_b, sem_sb, False)
    plsc.subcore_barrier()
    pltpu.sync_copy(num_sh.at[pl.ds(srow, npc)],
                    num_out.at[cid, pl.ds(srow, npc)])
    pltpu.sync_copy(den_sh.at[pl.ds(srow, npc)],
                    den_out.at[cid, pl.ds(srow, npc)])


def _edge_vals(currents2, targets2, act, cases_flat):
    mesh = plsc.VectorSubcoreMesh(core_axis_name="c", subcore_axis_name="s")
    f = pl.kernel(
        _stage1_body,
        out_type=jax.ShapeDtypeStruct((_E,), jnp.float32),
        mesh=mesh,
        compiler_params=pltpu.CompilerParams(needs_layout_passes=False,
                                             use_tc_tiling_on_sc=True),
        scratch_types=[
            pltpu.VMEM((_EPW // 128, 128), jnp.int32),
            pltpu.VMEM((_EPW // 128, 128), jnp.int32),
            pltpu.VMEM((_EPW,), jnp.float32),
            pltpu.VMEM((_S1, 2 * _D), jnp.float32),
            pltpu.VMEM((_S1, 2 * _D), jnp.float32),
            pltpu.VMEM((_S1, _D), jnp.float32),
            pltpu.VMEM((_S1, 2 * _D), jnp.float32),
            pltpu.VMEM((_S1, 2 * _D), jnp.float32),
            pltpu.VMEM((_S1, _D), jnp.float32),
            pltpu.VMEM((16 * 17,), jnp.float32),
            pltpu.SemaphoreType.DMA,
            pltpu.SemaphoreType.DMA,
        ],
    )
    return f(currents2, targets2, act, cases_flat)


def _segment_sums(currents2, targets2, v2, af):
    mesh = plsc.VectorSubcoreMesh(core_axis_name="c", subcore_axis_name="s")
    z2d = jnp.zeros((_N, _D), jnp.float32)
    z1d = jnp.zeros((_N,), jnp.float32)
    f = pl.kernel(
        _stage2_body,
        out_type=(jax.ShapeDtypeStruct((_NC, _N, _D), jnp.float32),
                  jax.ShapeDtypeStruct((_NC, _N), jnp.float32)),
        mesh=mesh,
        compiler_params=pltpu.CompilerParams(needs_layout_passes=False,
                                             use_tc_tiling_on_sc=False),
        scratch_types=[
            pltpu.VMEM((_EPW // 128, 128), jnp.int32),
            pltpu.VMEM((_EPW // 128, 128), jnp.int32),
            pltpu.VMEM((_EPW // 128, 128), jnp.float32),
            pltpu.VMEM((_S2, _D), jnp.float32),
            pltpu.VMEM((_S2, _D), jnp.float32),
            pltpu.VMEM_SHARED((_N, _D), jnp.float32),
            pltpu.VMEM_SHARED((_N,), jnp.float32),
            pltpu.VMEM_SHARED((_N, _D), jnp.float32),
            pltpu.SemaphoreType.DMA,
            pltpu.SemaphoreType.DMA,
            pltpu.SemaphoreType.DMA,
            pltpu.SemaphoreType.DMA,
        ],
    )
    return f(currents2, targets2, v2, af, z2d, z1d)


def kernel(currents, targets, activities_features, cases_features, W):
    c2 = currents.reshape(_E // 128, 128)
    t2 = targets.reshape(_E // 128, 128)
    af = _matmul_af(W, activities_features)
    act_pad = jnp.pad(activities_features, ((0, 0), (0, _D)))
    v = _edge_vals(c2, t2, act_pad, cases_features)
    num2, den2 = _segment_sums(c2, t2, v.reshape(_E // 128, 128), af)
    return _combine(num2, den2, af)
